# baseline (speedup 1.0000x reference)
import functools

import jax
import jax.numpy as jnp
from jax import lax
from jax.experimental import pallas as pl
from jax.experimental.pallas import tpu as pltpu
from jax.experimental.pallas import tpu_sc as plsc

_NUM_SPECIAL = 4
_NUM_FIXED = 100000
_NUM_LEARNED = 100000
_D = 64
_BATCH = 4096
_HIST = 50
_B = _BATCH * _HIST

_NC = 2
_NS = 16
_NW = _NC * _NS
_PER_W = _B // _NW
_CH = 128
_NCH = _PER_W // _CH

_LEARNED_START = _NUM_SPECIAL + _NUM_FIXED


def _compute_idx_chunk(ids_v, c, idxf_v, idxl_v):
    for k in range(_CH // 16):
        sl = pl.ds(k * 16, 16)
        idv = ids_v[pl.ds(c * _CH + k * 16, 16)]
        is_l = idv >= _LEARNED_START
        fi = jnp.maximum(idv - (_NUM_SPECIAL - 1), 0)
        fi = jnp.where(is_l, 0, fi)
        li = jnp.where(is_l, idv - (_LEARNED_START - 1), 0)
        idxf_v[sl] = fi
        idxl_v[sl] = li


def _add_chunk(dst_v, a_v, b_v):
    def body(i, _):
        for k in range(_D // 16):
            sl = pl.ds(k * 16, 16)
            dst_v[i, sl] = a_v[i, sl] + b_v[i, sl]
        return 0
    lax.fori_loop(0, _CH, body, 0)


def _emb_body(ids_hbm, fixed_hbm, learned_hbm, out_hbm,
              ids_v, idxf_v, idxl_v, rowf_v, rowl_v, sum_v,
              semf, seml, semo):
    cid = lax.axis_index("c")
    sid = lax.axis_index("s")
    wid = sid * _NC + cid
    base = wid * _PER_W

    pltpu.sync_copy(ids_hbm.at[pl.ds(base, _PER_W)], ids_v)

    def chunk(c, _):
        _compute_idx_chunk(ids_v, c, idxf_v, idxl_v)
        cf = pltpu.async_copy(fixed_hbm.at[idxf_v], rowf_v, semf)
        cl = pltpu.async_copy(learned_hbm.at[idxl_v], rowl_v, seml)
        cf.wait()
        cl.wait()
        _add_chunk(sum_v, rowf_v, rowl_v)
        pltpu.async_copy(
            sum_v, out_hbm.at[pl.ds(base + c * _CH, _CH)], semo
        ).wait()
        return 0

    lax.fori_loop(0, _NCH, chunk, 0)


@jax.jit
def _emb(ids_flat, fixed_table, learned_table):
    mesh = plsc.VectorSubcoreMesh(core_axis_name="c", subcore_axis_name="s")
    return pl.kernel(
        _emb_body,
        mesh=mesh,
        compiler_params=pltpu.CompilerParams(use_tc_tiling_on_sc=False),
        out_type=jax.ShapeDtypeStruct((_B, _D), jnp.float32),
        scratch_types=[
            pltpu.VMEM((_PER_W,), jnp.int32),
            pltpu.VMEM((_CH,), jnp.int32),
            pltpu.VMEM((_CH,), jnp.int32),
            pltpu.VMEM((_CH, _D), jnp.float32),
            pltpu.VMEM((_CH, _D), jnp.float32),
            pltpu.VMEM((_CH, _D), jnp.float32),
            pltpu.SemaphoreType.DMA,
            pltpu.SemaphoreType.DMA,
            pltpu.SemaphoreType.DMA,
        ],
    )(ids_flat, fixed_table, learned_table)


def kernel(ids_tensor, fixed_table, learned_table):
    ids_flat = ids_tensor.reshape(_B)
    out = _emb(ids_flat, fixed_table, learned_table)
    return out.reshape(_BATCH, _HIST, _D)


# speedup vs baseline: 1.0030x; 1.0030x over previous
import functools

import jax
import jax.numpy as jnp
from jax import lax
from jax.experimental import pallas as pl
from jax.experimental.pallas import tpu as pltpu
from jax.experimental.pallas import tpu_sc as plsc

_NUM_SPECIAL = 4
_NUM_FIXED = 100000
_NUM_LEARNED = 100000
_D = 64
_BATCH = 4096
_HIST = 50
_B = _BATCH * _HIST

_NC = 2
_NS = 16
_NW = _NC * _NS
_PER_W = _B // _NW
_CH = 128
_NCH = _PER_W // _CH
_NG = _NCH // 2

_LEARNED_START = _NUM_SPECIAL + _NUM_FIXED


def _emb_body(ids_hbm, fixed_hbm, learned_hbm, out_hbm,
              ids_v, idxf0, idxf1, idxl0, idxl1,
              rowf0, rowf1, rowl0, rowl1, sum0, sum1,
              semf0, semf1, seml0, seml1, semo0, semo1):
    cid = lax.axis_index("c")
    sid = lax.axis_index("s")
    wid = sid * _NC + cid
    base = wid * _PER_W

    idxf = [idxf0, idxf1]
    idxl = [idxl0, idxl1]
    rowf = [rowf0, rowf1]
    rowl = [rowl0, rowl1]
    sums = [sum0, sum1]
    semf = [semf0, semf1]
    seml = [seml0, seml1]
    semo = [semo0, semo1]

    pltpu.sync_copy(ids_hbm.at[pl.ds(base, _PER_W)], ids_v)

    def fire(c, b):
        for k in range(_CH // 16):
            sl = pl.ds(k * 16, 16)
            idv = ids_v[pl.ds(c * _CH + k * 16, 16)]
            is_l = idv >= _LEARNED_START
            fi = jnp.maximum(idv - (_NUM_SPECIAL - 1), 0)
            fi = jnp.where(is_l, 0, fi)
            li = jnp.where(is_l, idv - (_LEARNED_START - 1), 0)
            idxf[b][sl] = fi
            idxl[b][sl] = li
        pltpu.async_copy(fixed_hbm.at[idxf[b]], rowf[b], semf[b])
        pltpu.async_copy(learned_hbm.at[idxl[b]], rowl[b], seml[b])

    def wait_gathers(b):
        pltpu.make_async_copy(fixed_hbm.at[idxf[b]], rowf[b], semf[b]).wait()
        pltpu.make_async_copy(learned_hbm.at[idxl[b]], rowl[b], seml[b]).wait()

    def wait_out(c, b):
        pltpu.make_async_copy(
            sums[b], out_hbm.at[pl.ds(base + c * _CH, _CH)], semo[b]
        ).wait()

    def add_chunk(b):
        def body(i, _):
            for k in range(_D // 16):
                sl = pl.ds(k * 16, 16)
                sums[b][i, sl] = rowf[b][i, sl] + rowl[b][i, sl]
            return 0
        lax.fori_loop(0, _CH, body, 0)

    def put_out(c, b):
        pltpu.async_copy(
            sums[b], out_hbm.at[pl.ds(base + c * _CH, _CH)], semo[b]
        )

    fire(0, 0)
    fire(1, 1)
    for b in range(2):
        wait_gathers(b)
        add_chunk(b)
        put_out(b, b)
        fire(b + 2, b)

    def group(g, _):
        for b in range(2):
            c = g * 2 + b
            wait_gathers(b)
            wait_out(c - 2, b)
            add_chunk(b)
            put_out(c, b)
            fire(c + 2, b)
        return 0

    lax.fori_loop(1, _NG - 1, group, 0)

    for b in range(2):
        c = (_NG - 1) * 2 + b
        wait_gathers(b)
        wait_out(c - 2, b)
        add_chunk(b)
        put_out(c, b)
    for b in range(2):
        wait_out((_NG - 1) * 2 + b, b)


@jax.jit
def _emb(ids_flat, fixed_table, learned_table):
    mesh = plsc.VectorSubcoreMesh(core_axis_name="c", subcore_axis_name="s")
    return pl.kernel(
        _emb_body,
        mesh=mesh,
        compiler_params=pltpu.CompilerParams(use_tc_tiling_on_sc=False),
        out_type=jax.ShapeDtypeStruct((_B, _D), jnp.float32),
        scratch_types=[
            pltpu.VMEM((_PER_W,), jnp.int32),
            pltpu.VMEM((_CH,), jnp.int32),
            pltpu.VMEM((_CH,), jnp.int32),
            pltpu.VMEM((_CH,), jnp.int32),
            pltpu.VMEM((_CH,), jnp.int32),
            pltpu.VMEM((_CH, _D), jnp.float32),
            pltpu.VMEM((_CH, _D), jnp.float32),
            pltpu.VMEM((_CH, _D), jnp.float32),
            pltpu.VMEM((_CH, _D), jnp.float32),
            pltpu.VMEM((_CH, _D), jnp.float32),
            pltpu.VMEM((_CH, _D), jnp.float32),
            pltpu.SemaphoreType.DMA,
            pltpu.SemaphoreType.DMA,
            pltpu.SemaphoreType.DMA,
            pltpu.SemaphoreType.DMA,
            pltpu.SemaphoreType.DMA,
            pltpu.SemaphoreType.DMA,
        ],
    )(ids_flat, fixed_table, learned_table)


def kernel(ids_tensor, fixed_table, learned_table):
    ids_flat = ids_tensor.reshape(_B)
    out = _emb(ids_flat, fixed_table, learned_table)
    return out.reshape(_BATCH, _HIST, _D)


# speedup vs baseline: 2.5969x; 2.5893x over previous
import functools

import jax
import jax.numpy as jnp
from jax import lax
from jax.experimental import pallas as pl
from jax.experimental.pallas import tpu as pltpu
from jax.experimental.pallas import tpu_sc as plsc

_NUM_SPECIAL = 4
_NUM_FIXED = 100000
_NUM_LEARNED = 100000
_D = 64
_BATCH = 4096
_HIST = 50
_B = _BATCH * _HIST
_ROWS = _NUM_FIXED + 1

_NC = 2
_NS = 16
_NW = _NC * _NS
_PER_W = _B // _NW
_CH = 128
_NCH = _PER_W // _CH
_NG = _NCH // 2

_LEARNED_START = _NUM_SPECIAL + _NUM_FIXED

_BLK = 8192
_NB = -(-_ROWS // _BLK)


def _build_body(fixed_ref, learned_ref, l0_ref, f0_ref, out_ref):
    t = pl.program_id(0)
    out_ref[0] = jnp.where(
        t == 0,
        fixed_ref[...] + l0_ref[0],
        learned_ref[...] + f0_ref[0],
    )


def _build_combined(fixed_table, learned_table):
    l0 = learned_table[0:1]
    f0 = fixed_table[0:1]
    return pl.pallas_call(
        _build_body,
        grid=(2, _NB),
        in_specs=[
            pl.BlockSpec((_BLK, _D), lambda t, g: (jnp.where(t == 0, g, 0), 0)),
            pl.BlockSpec((_BLK, _D), lambda t, g: (jnp.where(t == 0, 0, g), 0)),
            pl.BlockSpec((1, _D), lambda t, g: (0, 0)),
            pl.BlockSpec((1, _D), lambda t, g: (0, 0)),
        ],
        out_specs=pl.BlockSpec((1, _BLK, _D), lambda t, g: (t, g, 0)),
        out_shape=jax.ShapeDtypeStruct((2, _ROWS, _D), jnp.float32),
    )(fixed_table, learned_table, l0, f0)


def _gather_body(ids_hbm, comb_hbm, out_hbm,
                 ids_v, idx0, idx1, idx2, rows0, rows1, rows2,
                 semg0, semg1, semg2, semo0, semo1, semo2):
    cid = lax.axis_index("c")
    sid = lax.axis_index("s")
    wid = sid * _NC + cid
    base = wid * _PER_W

    idx = [idx0, idx1, idx2]
    rows = [rows0, rows1, rows2]
    semg = [semg0, semg1, semg2]
    semo = [semo0, semo1, semo2]

    pltpu.sync_copy(ids_hbm.at[pl.ds(base, _PER_W)], ids_v)

    def fire(c, b):
        for k in range(_CH // 16):
            sl = pl.ds(k * 16, 16)
            idv = ids_v[pl.ds(c * _CH + k * 16, 16)]
            is_l = idv >= _LEARNED_START
            fi = jnp.maximum(idv - (_NUM_SPECIAL - 1), 0)
            ci = jnp.where(is_l, idv - 2, fi)
            idx[b][sl] = ci
        pltpu.async_copy(comb_hbm.at[idx[b]], rows[b], semg[b])

    def wait_gather(b):
        pltpu.make_async_copy(comb_hbm.at[idx[b]], rows[b], semg[b]).wait()

    def put_out(c, b):
        pltpu.async_copy(
            rows[b], out_hbm.at[pl.ds(base + c * _CH, _CH)], semo[b]
        )

    def wait_out(c, b):
        pltpu.make_async_copy(
            rows[b], out_hbm.at[pl.ds(base + c * _CH, _CH)], semo[b]
        ).wait()

    fire(0, 0)
    fire(1, 1)

    fire(2, 2)
    wait_gather(0)
    put_out(0, 0)
    wait_out(0, 0)
    fire(3, 0)
    wait_gather(1)
    put_out(1, 1)
    wait_out(1, 1)
    fire(4, 1)
    wait_gather(2)
    put_out(2, 2)

    def group(g, _):
        for j in range(3):
            c = g * 3 + j
            bn = (j + 2) % 3
            wait_out(c - 1, bn)
            fire(c + 2, bn)
            wait_gather(j)
            put_out(c, j)
        return 0

    lax.fori_loop(1, _NCH // 3, group, 0)

    wait_out(47, 2)
    wait_gather(0)
    put_out(48, 0)
    wait_gather(1)
    put_out(49, 1)
    wait_out(48, 0)
    wait_out(49, 1)


@jax.jit
def _emb(ids_flat, fixed_table, learned_table):
    comb = _build_combined(fixed_table, learned_table).reshape(2 * _ROWS, _D)
    mesh = plsc.VectorSubcoreMesh(core_axis_name="c", subcore_axis_name="s")
    return pl.kernel(
        _gather_body,
        mesh=mesh,
        compiler_params=pltpu.CompilerParams(use_tc_tiling_on_sc=False),
        out_type=jax.ShapeDtypeStruct((_B, _D), jnp.float32),
        scratch_types=[
            pltpu.VMEM((_PER_W,), jnp.int32),
            pltpu.VMEM((_CH,), jnp.int32),
            pltpu.VMEM((_CH,), jnp.int32),
            pltpu.VMEM((_CH,), jnp.int32),
            pltpu.VMEM((_CH, _D), jnp.float32),
            pltpu.VMEM((_CH, _D), jnp.float32),
            pltpu.VMEM((_CH, _D), jnp.float32),
            pltpu.SemaphoreType.DMA,
            pltpu.SemaphoreType.DMA,
            pltpu.SemaphoreType.DMA,
            pltpu.SemaphoreType.DMA,
            pltpu.SemaphoreType.DMA,
            pltpu.SemaphoreType.DMA,
        ],
    )(ids_flat, comb)


def kernel(ids_tensor, fixed_table, learned_table):
    ids_flat = ids_tensor.reshape(_B)
    out = _emb(ids_flat, fixed_table, learned_table)
    return out.reshape(_BATCH, _HIST, _D)


# speedup vs baseline: 5.6628x; 2.1806x over previous
import functools

import jax
import jax.numpy as jnp
from jax import lax
from jax.experimental import pallas as pl
from jax.experimental.pallas import tpu as pltpu
from jax.experimental.pallas import tpu_sc as plsc

_NUM_SPECIAL = 4
_NUM_FIXED = 100000
_NUM_LEARNED = 100000
_D = 64
_BATCH = 4096
_HIST = 50
_B = _BATCH * _HIST
_ROWS = _NUM_FIXED + 1

_NC = 2
_NS = 16
_NW = _NC * _NS
_PER_W = _B // _NW
_CH = 128
_NCH = _PER_W // _CH
_NG = _NCH // 2

_LEARNED_START = _NUM_SPECIAL + _NUM_FIXED

_BLK = 8192
_NBF = -(-_ROWS // _BLK)
_L_OFF = _NBF * _BLK
_CROWS = 2 * _L_OFF


def _build_body(fixed_ref, learned_ref, l0_ref, f0_ref, out_ref):
    g = pl.program_id(0)
    out_ref[...] = jnp.where(
        g < _NBF,
        fixed_ref[...] + l0_ref[0],
        learned_ref[...] + f0_ref[0],
    )


def _build_combined(fixed_table, learned_table):
    l0 = learned_table[0:1]
    f0 = fixed_table[0:1]
    return pl.pallas_call(
        _build_body,
        grid=(2 * _NBF,),
        in_specs=[
            pl.BlockSpec((_BLK, _D), lambda g: (jnp.where(g < _NBF, g, 0), 0)),
            pl.BlockSpec((_BLK, _D), lambda g: (jnp.where(g < _NBF, 0, g - _NBF), 0)),
            pl.BlockSpec((1, _D), lambda g: (0, 0)),
            pl.BlockSpec((1, _D), lambda g: (0, 0)),
        ],
        out_specs=pl.BlockSpec((_BLK, _D), lambda g: (g, 0)),
        out_shape=jax.ShapeDtypeStruct((_CROWS, _D), jnp.float32),
    )(fixed_table, learned_table, l0, f0)


def _gather_body(ids_hbm, comb_hbm, out_hbm,
                 ids_v, idx0, idx1, idx2, rows0, rows1, rows2,
                 semg0, semg1, semg2, semo0, semo1, semo2):
    cid = lax.axis_index("c")
    sid = lax.axis_index("s")
    wid = sid * _NC + cid
    base = wid * _PER_W

    idx = [idx0, idx1, idx2]
    rows = [rows0, rows1, rows2]
    semg = [semg0, semg1, semg2]
    semo = [semo0, semo1, semo2]

    pltpu.sync_copy(ids_hbm.at[pl.ds(base, _PER_W)], ids_v)

    def fire(c, b):
        for k in range(_CH // 16):
            sl = pl.ds(k * 16, 16)
            idv = ids_v[pl.ds(c * _CH + k * 16, 16)]
            is_l = idv >= _LEARNED_START
            fi = jnp.maximum(idv - (_NUM_SPECIAL - 1), 0)
            ci = jnp.where(is_l, idv + (_L_OFF - (_LEARNED_START - 1)), fi)
            idx[b][sl] = ci
        pltpu.async_copy(comb_hbm.at[idx[b]], rows[b], semg[b])

    def wait_gather(b):
        pltpu.make_async_copy(comb_hbm.at[idx[b]], rows[b], semg[b]).wait()

    def put_out(c, b):
        pltpu.async_copy(
            rows[b], out_hbm.at[pl.ds(base + c * _CH, _CH)], semo[b]
        )

    def wait_out(c, b):
        pltpu.make_async_copy(
            rows[b], out_hbm.at[pl.ds(base + c * _CH, _CH)], semo[b]
        ).wait()

    fire(0, 0)
    fire(1, 1)

    fire(2, 2)
    wait_gather(0)
    put_out(0, 0)
    wait_out(0, 0)
    fire(3, 0)
    wait_gather(1)
    put_out(1, 1)
    wait_out(1, 1)
    fire(4, 1)
    wait_gather(2)
    put_out(2, 2)

    def group(g, _):
        for j in range(3):
            c = g * 3 + j
            bn = (j + 2) % 3
            wait_out(c - 1, bn)
            fire(c + 2, bn)
            wait_gather(j)
            put_out(c, j)
        return 0

    lax.fori_loop(1, _NCH // 3, group, 0)

    wait_out(47, 2)
    wait_gather(0)
    put_out(48, 0)
    wait_gather(1)
    put_out(49, 1)
    wait_out(48, 0)
    wait_out(49, 1)


@jax.jit
def _emb(ids_flat, fixed_table, learned_table):
    comb = _build_combined(fixed_table, learned_table)
    mesh = plsc.VectorSubcoreMesh(core_axis_name="c", subcore_axis_name="s")
    return pl.kernel(
        _gather_body,
        mesh=mesh,
        compiler_params=pltpu.CompilerParams(use_tc_tiling_on_sc=False),
        out_type=jax.ShapeDtypeStruct((_B, _D), jnp.float32),
        scratch_types=[
            pltpu.VMEM((_PER_W,), jnp.int32),
            pltpu.VMEM((_CH,), jnp.int32),
            pltpu.VMEM((_CH,), jnp.int32),
            pltpu.VMEM((_CH,), jnp.int32),
            pltpu.VMEM((_CH, _D), jnp.float32),
            pltpu.VMEM((_CH, _D), jnp.float32),
            pltpu.VMEM((_CH, _D), jnp.float32),
            pltpu.SemaphoreType.DMA,
            pltpu.SemaphoreType.DMA,
            pltpu.SemaphoreType.DMA,
            pltpu.SemaphoreType.DMA,
            pltpu.SemaphoreType.DMA,
            pltpu.SemaphoreType.DMA,
        ],
    )(ids_flat, comb)


def kernel(ids_tensor, fixed_table, learned_table):
    ids_flat = ids_tensor.reshape(_B)
    out = _emb(ids_flat, fixed_table, learned_table)
    return out.reshape(_BATCH, _HIST, _D)


# speedup vs baseline: 5.6712x; 1.0015x over previous
import functools

import jax
import jax.numpy as jnp
from jax import lax
from jax.experimental import layout as jex_layout
from jax.experimental import pallas as pl
from jax.experimental.pallas import tpu as pltpu
from jax.experimental.pallas import tpu_sc as plsc

_NUM_SPECIAL = 4
_NUM_FIXED = 100000
_NUM_LEARNED = 100000
_D = 64
_BATCH = 4096
_HIST = 50
_B = _BATCH * _HIST
_ROWS = _NUM_FIXED + 1

_NC = 2
_NS = 16
_NW = _NC * _NS
_PER_W = _B // _NW
_CH = 128
_NCH = _PER_W // _CH
_NG = _NCH // 2

_LEARNED_START = _NUM_SPECIAL + _NUM_FIXED

_BLK = 8192
_NBF = -(-_ROWS // _BLK)
_L_OFF = _NBF * _BLK
_CROWS = 2 * _L_OFF


def _build_body(fixed_ref, learned_ref, l0_ref, f0_ref, out_ref):
    g = pl.program_id(0)
    out_ref[...] = jnp.where(
        g < _NBF,
        fixed_ref[...] + l0_ref[0],
        learned_ref[...] + f0_ref[0],
    )


def _build_combined(fixed_table, learned_table):
    l0 = learned_table[0:1]
    f0 = fixed_table[0:1]
    return pl.pallas_call(
        _build_body,
        grid=(2 * _NBF,),
        in_specs=[
            pl.BlockSpec((_BLK, _D), lambda g: (jnp.where(g < _NBF, g, 0), 0)),
            pl.BlockSpec((_BLK, _D), lambda g: (jnp.where(g < _NBF, 0, g - _NBF), 0)),
            pl.BlockSpec((1, _D), lambda g: (0, 0)),
            pl.BlockSpec((1, _D), lambda g: (0, 0)),
        ],
        out_specs=pl.BlockSpec((_BLK, _D), lambda g: (g, 0)),
        out_shape=jax.ShapeDtypeStruct((_CROWS, _D), jnp.float32),
    )(fixed_table, learned_table, l0, f0)


def _gather_body(ids_hbm, comb_hbm, out_hbm,
                 ids_v, idx0, idx1, idx2, rows0, rows1, rows2,
                 semg0, semg1, semg2, semo0, semo1, semo2):
    cid = lax.axis_index("c")
    sid = lax.axis_index("s")
    wid = sid * _NC + cid
    base = wid * _PER_W

    idx = [idx0, idx1, idx2]
    rows = [rows0, rows1, rows2]
    semg = [semg0, semg1, semg2]
    semo = [semo0, semo1, semo2]

    pltpu.sync_copy(ids_hbm.at[pl.ds(base, _PER_W)], ids_v)

    def fire(c, b):
        for k in range(_CH // 16):
            sl = pl.ds(k * 16, 16)
            idv = ids_v[pl.ds(c * _CH + k * 16, 16)]
            is_l = idv >= _LEARNED_START
            fi = jnp.maximum(idv - (_NUM_SPECIAL - 1), 0)
            ci = jnp.where(is_l, idv + (_L_OFF - (_LEARNED_START - 1)), fi)
            idx[b][sl] = ci
        pltpu.async_copy(comb_hbm.at[idx[b]], rows[b], semg[b])

    def wait_gather(b):
        pltpu.make_async_copy(comb_hbm.at[idx[b]], rows[b], semg[b]).wait()

    def put_out(c, b):
        pltpu.async_copy(
            rows[b], out_hbm.at[pl.ds(base + c * _CH, _CH)], semo[b]
        )

    def wait_out(c, b):
        pltpu.make_async_copy(
            rows[b], out_hbm.at[pl.ds(base + c * _CH, _CH)], semo[b]
        ).wait()

    fire(0, 0)
    fire(1, 1)

    fire(2, 2)
    wait_gather(0)
    put_out(0, 0)
    wait_out(0, 0)
    fire(3, 0)
    wait_gather(1)
    put_out(1, 1)
    wait_out(1, 1)
    fire(4, 1)
    wait_gather(2)
    put_out(2, 2)

    def group(g, _):
        for j in range(3):
            c = g * 3 + j
            bn = (j + 2) % 3
            wait_out(c - 1, bn)
            fire(c + 2, bn)
            wait_gather(j)
            put_out(c, j)
        return 0

    lax.fori_loop(1, _NCH // 3, group, 0)

    wait_out(47, 2)
    wait_gather(0)
    put_out(48, 0)
    wait_gather(1)
    put_out(49, 1)
    wait_out(48, 0)
    wait_out(49, 1)


def _emb(ids_flat, fixed_table, learned_table):
    comb = _build_combined(fixed_table, learned_table)
    mesh = plsc.VectorSubcoreMesh(core_axis_name="c", subcore_axis_name="s")
    out = pl.kernel(
        _gather_body,
        mesh=mesh,
        compiler_params=pltpu.CompilerParams(use_tc_tiling_on_sc=False),
        out_type=jax.ShapeDtypeStruct((_B, _D), jnp.float32),
        scratch_types=[
            pltpu.VMEM((_PER_W,), jnp.int32),
            pltpu.VMEM((_CH,), jnp.int32),
            pltpu.VMEM((_CH,), jnp.int32),
            pltpu.VMEM((_CH,), jnp.int32),
            pltpu.VMEM((_CH, _D), jnp.float32),
            pltpu.VMEM((_CH, _D), jnp.float32),
            pltpu.VMEM((_CH, _D), jnp.float32),
            pltpu.SemaphoreType.DMA,
            pltpu.SemaphoreType.DMA,
            pltpu.SemaphoreType.DMA,
            pltpu.SemaphoreType.DMA,
            pltpu.SemaphoreType.DMA,
            pltpu.SemaphoreType.DMA,
        ],
    )(ids_flat, comb)
    return out.reshape(_BATCH, _HIST, _D)


@functools.lru_cache(maxsize=None)
def _jitted_emb(device):
    fmt = jex_layout.Format(
        jex_layout.Layout(major_to_minor=(0, 1, 2), tiling=((8,),)),
        jax.sharding.SingleDeviceSharding(device),
    )
    return jax.jit(_emb, out_shardings=fmt)


def kernel(ids_tensor, fixed_table, learned_table):
    ids_flat = ids_tensor.reshape(_B)
    return _jitted_emb(jax.devices()[0])(ids_flat, fixed_table, learned_table)


# speedup vs baseline: 5.6969x; 1.0045x over previous
import functools

import jax
import jax.numpy as jnp
from jax import lax
from jax.experimental import layout as jex_layout
from jax.experimental import pallas as pl
from jax.experimental.pallas import tpu as pltpu
from jax.experimental.pallas import tpu_sc as plsc

_NUM_SPECIAL = 4
_NUM_FIXED = 100000
_NUM_LEARNED = 100000
_D = 64
_BATCH = 4096
_HIST = 50
_B = _BATCH * _HIST
_ROWS = _NUM_FIXED + 1

_NC = 2
_NS = 16
_NW = _NC * _NS
_PER_W = _B // _NW
_CH = 128
_NCH = _PER_W // _CH
_NG = _NCH // 2

_LEARNED_START = _NUM_SPECIAL + _NUM_FIXED

_BLK = 8192
_NBF = -(-_ROWS // _BLK)
_L_OFF = _NBF * _BLK
_CROWS = 2 * _L_OFF


def _build_body(fixed_ref, learned_ref, l0_ref, f0_ref, out_ref):
    g = pl.program_id(0)
    out_ref[...] = jnp.where(
        g < _NBF,
        fixed_ref[...] + l0_ref[0],
        learned_ref[...] + f0_ref[0],
    )


def _build_combined(fixed_table, learned_table):
    return pl.pallas_call(
        _build_body,
        grid=(2 * _NBF,),
        in_specs=[
            pl.BlockSpec((_BLK, _D), lambda g: (jnp.where(g < _NBF, g, 0), 0)),
            pl.BlockSpec((_BLK, _D), lambda g: (jnp.where(g < _NBF, 0, g - _NBF), 0)),
            pl.BlockSpec((8, _D), lambda g: (0, 0)),
            pl.BlockSpec((8, _D), lambda g: (0, 0)),
        ],
        out_specs=pl.BlockSpec((_BLK, _D), lambda g: (g, 0)),
        out_shape=jax.ShapeDtypeStruct((_CROWS, _D), jnp.float32),
    )(fixed_table, learned_table, learned_table, fixed_table)


_GRP = 400
_GB = _GRP // _HIST
_NGRP = _PER_W // _GRP
_GCH = (128, 128, 128, 16)


def _gather_body(ids_hbm, comb_hbm, out_hbm,
                 ids_v, idx0, idx1, rows0, rows1,
                 semg0, semg1, semo0, semo1):
    cid = lax.axis_index("c")
    sid = lax.axis_index("s")
    wid = sid * _NC + cid
    base = wid * _PER_W
    bat0 = wid * (_PER_W // _HIST)

    idx = [idx0, idx1]
    rows = [rows0, rows1]
    semg = [semg0, semg1]
    semo = [semo0, semo1]

    pltpu.sync_copy(ids_hbm.at[pl.ds(base, _PER_W)], ids_v)

    def fire(g, b):
        for k in range(_GRP // 16):
            sl = pl.ds(k * 16, 16)
            idv = ids_v[pl.ds(g * _GRP + k * 16, 16)]
            is_l = idv >= _LEARNED_START
            fi = jnp.maximum(idv - (_NUM_SPECIAL - 1), 0)
            ci = jnp.where(is_l, idv + (_L_OFF - (_LEARNED_START - 1)), fi)
            idx[b][sl] = ci
        off = 0
        for n in _GCH:
            pltpu.async_copy(
                comb_hbm.at[idx[b].at[pl.ds(off, n)]],
                rows[b].at[pl.ds(off, n)], semg[b])
            off += n

    def wait_gathers(b):
        off = 0
        for n in _GCH:
            pltpu.make_async_copy(
                comb_hbm.at[idx[b].at[pl.ds(off, n)]],
                rows[b].at[pl.ds(off, n)], semg[b]).wait()
            off += n

    def put_outs(g, b):
        for j in range(_GB):
            pltpu.async_copy(
                rows[b].at[pl.ds(j * _HIST, _HIST)],
                out_hbm.at[bat0 + g * _GB + j], semo[b])

    def wait_outs(g, b):
        for j in range(_GB):
            pltpu.make_async_copy(
                rows[b].at[pl.ds(j * _HIST, _HIST)],
                out_hbm.at[bat0 + g * _GB + j], semo[b]).wait()

    fire(0, 0)
    for g in range(_NGRP):
        b = g % 2
        if g + 1 < _NGRP:
            if g >= 1:
                wait_outs(g - 1, (g + 1) % 2)
            fire(g + 1, (g + 1) % 2)
        wait_gathers(b)
        put_outs(g, b)
    wait_outs(_NGRP - 2, _NGRP % 2)
    wait_outs(_NGRP - 1, (_NGRP - 1) % 2)


def _emb(ids_flat, fixed_table, learned_table):
    comb = _build_combined(fixed_table, learned_table)
    mesh = plsc.VectorSubcoreMesh(core_axis_name="c", subcore_axis_name="s")
    out = pl.kernel(
        _gather_body,
        mesh=mesh,
        compiler_params=pltpu.CompilerParams(use_tc_tiling_on_sc=False),
        out_type=jax.ShapeDtypeStruct((_BATCH, _HIST, _D), jnp.float32),
        scratch_types=[
            pltpu.VMEM((_PER_W,), jnp.int32),
            pltpu.VMEM((_GRP,), jnp.int32),
            pltpu.VMEM((_GRP,), jnp.int32),
            pltpu.VMEM((_GRP, _D), jnp.float32),
            pltpu.VMEM((_GRP, _D), jnp.float32),
            pltpu.SemaphoreType.DMA,
            pltpu.SemaphoreType.DMA,
            pltpu.SemaphoreType.DMA,
            pltpu.SemaphoreType.DMA,
        ],
    )(ids_flat, comb)
    return out


@functools.lru_cache(maxsize=None)
def _jitted_emb(device):
    fmt = jex_layout.Format(
        jex_layout.Layout(major_to_minor=(0, 1, 2), tiling=((8,),)),
        jax.sharding.SingleDeviceSharding(device),
    )
    return jax.jit(_emb, out_shardings=fmt)


def kernel(ids_tensor, fixed_table, learned_table):
    ids_flat = ids_tensor.reshape(_B)
    return _jitted_emb(jax.devices()[0])(ids_flat, fixed_table, learned_table)
